# baseline (speedup 1.0000x reference)
import functools

import jax
import jax.numpy as jnp
from jax import lax
from jax.experimental import pallas as pl
from jax.experimental.pallas import tpu as pltpu
from jax.experimental.pallas import tpu_sc as plsc

NC = 2
NS = 16
NW = NC * NS
CHUNK = 128
K = 4
SB = 4


def _mesh():
    return plsc.VectorSubcoreMesh(core_axis_name="c", subcore_axis_name="s")


def _sc_degree(n_pad, chunks):
    rpt = n_pad // NS
    nzc = rpt // CHUNK

    @functools.partial(
        pl.kernel,
        out_type=[jax.ShapeDtypeStruct((n_pad,), jnp.float32),
                  jax.ShapeDtypeStruct((n_pad,), jnp.float32)],
        mesh=_mesh(),
        scratch_types=[
            pltpu.VMEM((chunks, CHUNK), jnp.int32),
            pltpu.VMEM((CHUNK,), jnp.float32),
            pltpu.VMEM((CHUNK,), jnp.float32),
            pltpu.VMEM_SHARED((n_pad,), jnp.float32),
        ],
    )
    def deg_kernel(dst_hbm, out0, out1, idx_v, ones_v, stg_v, acc_sh):
        c = lax.axis_index("c")
        s = lax.axis_index("s")
        wid = s * NC + c
        for k in range(CHUNK // 16):
            ones_v[pl.ds(k * 16, 16)] = jnp.ones((16,), jnp.float32)
            stg_v[pl.ds(k * 16, 16)] = jnp.zeros((16,), jnp.float32)
        for i in range(nzc):
            pltpu.sync_copy(stg_v, acc_sh.at[pl.ds(s * rpt + i * CHUNK,
                                                   CHUNK)])
        plsc.subcore_barrier()
        pltpu.sync_copy(dst_hbm.at[wid], idx_v)

        def body(j, carry):
            pltpu.sync_copy(ones_v, acc_sh.at[idx_v.at[j]], add=True)
            return carry

        lax.fori_loop(0, chunks, body, 0)
        plsc.subcore_barrier()
        for i in range(nzc):
            sl = pl.ds(s * rpt + i * CHUNK, CHUNK)
            pltpu.sync_copy(acc_sh.at[sl], stg_v)

            @pl.when(c == 0)
            def _():
                pltpu.sync_copy(stg_v, out0.at[sl])

            @pl.when(c == 1)
            def _():
                pltpu.sync_copy(stg_v, out1.at[sl])

    return deg_kernel


def _sc_rows(n_pad, chunks, h):
    rpt = n_pad // NS
    nzc = rpt // CHUNK
    nblocks = chunks // SB

    @functools.partial(
        pl.kernel,
        out_type=jax.ShapeDtypeStruct((NC, n_pad, h), jnp.float32),
        mesh=_mesh(),
        scratch_types=[
            pltpu.VMEM((2, 2 * SB, CHUNK), jnp.int32),
            pltpu.VMEM((CHUNK, h), jnp.float32),
            pltpu.VMEM((CHUNK, h), jnp.float32),
            pltpu.VMEM_SHARED((n_pad, h), jnp.float32),
            pltpu.SemaphoreType.DMA((2,)),
            pltpu.SemaphoreType.DMA,
            pltpu.SemaphoreType.DMA,
        ],
    )
    def rows_kernel(y_hbm, comb_hbm, zeros_hbm, out_hbm,
                    ring, d0, d1, acc_sh, isem, gsem0, gsem1):
        dbufs = (d0, d1)
        gsems = (gsem0, gsem1)
        c = lax.axis_index("c")
        s = lax.axis_index("s")
        wid = s * NC + c
        pltpu.sync_copy(zeros_hbm, d0)
        for i in range(nzc):
            pltpu.sync_copy(d0,
                            acc_sh.at[pl.ds(s * rpt + i * CHUNK, CHUNK)])
        plsc.subcore_barrier()
        pltpu.sync_copy(comb_hbm.at[wid, pl.ds(0, 2 * SB)], ring.at[0])
        pltpu.async_copy(comb_hbm.at[wid, pl.ds(2 * SB, 2 * SB)],
                         ring.at[1], isem.at[1])
        pltpu.async_copy(y_hbm.at[ring.at[0, 0]], d0, gsem0)
        pltpu.async_copy(y_hbm.at[ring.at[0, 2]], d1, gsem1)

        def outer(o, carry):
            r = o % 2
            rn = (o + 1) % 2
            for i in range(SB):
                j = 4 * o + i
                d = i % 2
                pltpu.make_async_copy(y_hbm.at[ring.at[r, 2 * i]],
                                      dbufs[d], gsems[d]).wait()
                pltpu.sync_copy(dbufs[d],
                                acc_sh.at[ring.at[r, 2 * i + 1]], add=True)

                @pl.when(j + 2 < chunks)
                def _():
                    if i < 2:
                        pltpu.async_copy(
                            y_hbm.at[ring.at[r, 2 * (i + 2)]],
                            dbufs[d], gsems[d])
                    else:
                        if i == 2:
                            pltpu.make_async_copy(
                                comb_hbm.at[wid,
                                            pl.ds(2 * SB * (o + 1), 2 * SB)],
                                ring.at[rn], isem.at[rn]).wait()
                        pltpu.async_copy(
                            y_hbm.at[ring.at[rn, 2 * (i - 2)]],
                            dbufs[d], gsems[d])

            @pl.when(o + 2 < nblocks)
            def _():
                pltpu.async_copy(
                    comb_hbm.at[wid, pl.ds(2 * SB * (o + 2), 2 * SB)],
                    ring.at[r], isem.at[r])
            return carry

        lax.fori_loop(0, nblocks, outer, 0)
        plsc.subcore_barrier()
        for i in range(nzc):
            sl = pl.ds(s * rpt + i * CHUNK, CHUNK)
            pltpu.sync_copy(acc_sh.at[sl], d0)
            pltpu.sync_copy(d0, out_hbm.at[c, sl])

    return rows_kernel


def _sc_scalar(n_pad, chunks):
    rpt = n_pad // NS
    nzc = rpt // CHUNK
    niter = chunks // K

    @functools.partial(
        pl.kernel,
        out_type=[jax.ShapeDtypeStruct((n_pad,), jnp.float32),
                  jax.ShapeDtypeStruct((n_pad,), jnp.float32)],
        mesh=_mesh(),
        scratch_types=(
            [pltpu.VMEM((chunks, CHUNK), jnp.int32),
             pltpu.VMEM((chunks, CHUNK), jnp.int32)]
            + [pltpu.VMEM((CHUNK,), jnp.float32)] * K
            + [pltpu.VMEM_SHARED((n_pad,), jnp.float32)]
            + [pltpu.SemaphoreType.DMA] * K
        ),
    )
    def scal_kernel(y_hbm, src_hbm, dst_hbm, out0, out1,
                    src_v, dst_v, *bufsem):
        bufs = bufsem[:K]
        acc_sh = bufsem[K]
        gsem = bufsem[K + 1:]
        c = lax.axis_index("c")
        s = lax.axis_index("s")
        wid = s * NC + c
        for k in range(CHUNK // 16):
            bufs[0][pl.ds(k * 16, 16)] = jnp.zeros((16,), jnp.float32)
        for i in range(nzc):
            pltpu.sync_copy(bufs[0],
                            acc_sh.at[pl.ds(s * rpt + i * CHUNK, CHUNK)])
        plsc.subcore_barrier()
        pltpu.sync_copy(src_hbm.at[wid], src_v)
        pltpu.sync_copy(dst_hbm.at[wid], dst_v)

        for b in range(K):
            pltpu.async_copy(y_hbm.at[src_v.at[b]], bufs[b], gsem[b])

        def body(o, carry):
            for b in range(K):
                j = o * K + b
                pltpu.make_async_copy(y_hbm.at[src_v.at[j]],
                                      bufs[b], gsem[b]).wait()
                pltpu.sync_copy(bufs[b], acc_sh.at[dst_v.at[j]], add=True)

                @pl.when(o < niter - 1)
                def _():
                    pltpu.async_copy(y_hbm.at[src_v.at[j + K]],
                                     bufs[b], gsem[b])
            return carry

        lax.fori_loop(0, niter, body, 0)
        plsc.subcore_barrier()
        for i in range(nzc):
            sl = pl.ds(s * rpt + i * CHUNK, CHUNK)
            pltpu.sync_copy(acc_sh.at[sl], bufs[0])

            @pl.when(c == 0)
            def _():
                pltpu.sync_copy(bufs[0], out0.at[sl])

            @pl.when(c == 1)
            def _():
                pltpu.sync_copy(bufs[0], out1.at[sl])

    return scal_kernel


def _tc_prep(n_pad, f_in, h):
    def body(deg0_ref, deg1_ref, x_ref, w_ref, y_ref, dinv_ref):
        total = deg0_ref[...] + deg1_ref[...] + 1.0
        dinv = lax.rsqrt(total)
        dinv_ref[...] = dinv
        y_ref[...] = jnp.dot(x_ref[...], w_ref[...],
                             preferred_element_type=jnp.float32) * dinv

    return pl.pallas_call(
        body,
        out_shape=[
            jax.ShapeDtypeStruct((n_pad, h), jnp.float32),
            jax.ShapeDtypeStruct((n_pad, 1), jnp.float32),
        ],
    )


def _tc_mid(n, n_pad, h, w_out):
    def body(s_ref, y_ref, dinv_ref, b_ref, g_ref, be_ref, w_ref, out_ref):
        dinv = dinv_ref[...]
        pre = dinv * (s_ref[0] + s_ref[1] + y_ref[...]) + b_ref[...]
        row = lax.broadcasted_iota(jnp.int32, (n_pad, 1), 0)
        mask = row < n
        pm = jnp.where(mask, pre, 0.0)
        mean = jnp.sum(pm, axis=0, keepdims=True) * (1.0 / n)
        meansq = jnp.sum(pm * pre, axis=0, keepdims=True) * (1.0 / n)
        var = meansq - mean * mean
        hh = (pre - mean) * lax.rsqrt(var + 1e-5) * g_ref[...] + be_ref[...]
        hh = jnp.where(mask, jnp.maximum(hh, 0.0), 0.0)
        out_ref[...] = jnp.dot(hh, w_ref[...],
                               preferred_element_type=jnp.float32) * dinv

    return pl.pallas_call(
        body,
        out_shape=jax.ShapeDtypeStruct((n_pad, w_out), jnp.float32),
    )


def _tc_final(n_pad):
    def body(s0_ref, s1_ref, y_ref, dinv_ref, b_ref, out_ref):
        out_ref[...] = (dinv_ref[...] * (s0_ref[...] + s1_ref[...]
                                         + y_ref[...]) + b_ref[...])

    return pl.pallas_call(
        body,
        out_shape=jax.ShapeDtypeStruct((n_pad, 1), jnp.float32),
    )


def kernel(x, edge_index, W1, b1, g1, be1, W2, b2, g2, be2, W3, b3):
    n, f_in = x.shape
    h = W1.shape[1]
    e = edge_index.shape[1]

    cpw = -(-e // (NW * CHUNK))
    chunks = -(-cpw // SB) * SB
    e_pad = NW * chunks * CHUNK
    n_pad = -(-(n + 1) // (NS * CHUNK)) * (NS * CHUNK)

    src = jnp.concatenate(
        [edge_index[0].astype(jnp.int32),
         jnp.full((e_pad - e,), n, jnp.int32)]).reshape(NW, chunks, CHUNK)
    dst = jnp.concatenate(
        [edge_index[1].astype(jnp.int32),
         jnp.full((e_pad - e,), n, jnp.int32)]).reshape(NW, chunks, CHUNK)
    comb = jnp.stack([src, dst], axis=2).reshape(NW, 2 * chunks, CHUNK)
    x_p = jnp.zeros((n_pad, f_in), jnp.float32).at[:n].set(x)
    zeros2 = jnp.zeros((CHUNK, h), jnp.float32)

    deg0, deg1 = _sc_degree(n_pad, chunks)(dst)
    y1, dinv = _tc_prep(n_pad, f_in, h)(
        deg0.reshape(n_pad, 1), deg1.reshape(n_pad, 1), x_p, W1)

    s1 = _sc_rows(n_pad, chunks, h)(y1, comb, zeros2)
    y2 = _tc_mid(n, n_pad, h, h)(
        s1, y1, dinv, b1.reshape(1, h), g1.reshape(1, h),
        be1.reshape(1, h), W2)

    s2 = _sc_rows(n_pad, chunks, h)(y2, comb, zeros2)
    y3 = _tc_mid(n, n_pad, h, 1)(
        s2, y2, dinv, b2.reshape(1, h), g2.reshape(1, h),
        be2.reshape(1, h), W3)

    s3_0, s3_1 = _sc_scalar(n_pad, chunks)(y3.reshape(n_pad), src, dst)
    out = _tc_final(n_pad)(
        s3_0.reshape(n_pad, 1), s3_1.reshape(n_pad, 1), y3, dinv,
        b3.reshape(1, 1))
    return out[:n]


# speedup vs baseline: 2.7426x; 2.7426x over previous
import functools

import jax
import jax.numpy as jnp
from jax import lax
from jax.experimental import pallas as pl
from jax.experimental.pallas import tpu as pltpu
from jax.experimental.pallas import tpu_sc as plsc

NC = 2
NS = 16
NW = NC * NS
CHUNK = 128
K = 4
SB = 4


def _mesh():
    return plsc.VectorSubcoreMesh(core_axis_name="c", subcore_axis_name="s")


def _sc_degree(n_pad, chunks):
    rpt = n_pad // NS
    nzc = rpt // CHUNK

    @functools.partial(
        pl.kernel,
        out_type=[jax.ShapeDtypeStruct((n_pad,), jnp.float32),
                  jax.ShapeDtypeStruct((n_pad,), jnp.float32)],
        mesh=_mesh(),
        scratch_types=[
            pltpu.VMEM((chunks, CHUNK), jnp.int32),
            pltpu.VMEM((CHUNK,), jnp.float32),
            pltpu.VMEM((CHUNK,), jnp.float32),
            pltpu.VMEM_SHARED((n_pad,), jnp.float32),
        ],
    )
    def deg_kernel(dst_hbm, out0, out1, idx_v, ones_v, stg_v, acc_sh):
        c = lax.axis_index("c")
        s = lax.axis_index("s")
        wid = s * NC + c
        for k in range(CHUNK // 16):
            ones_v[pl.ds(k * 16, 16)] = jnp.ones((16,), jnp.float32)
            stg_v[pl.ds(k * 16, 16)] = jnp.zeros((16,), jnp.float32)
        for i in range(nzc):
            pltpu.sync_copy(stg_v, acc_sh.at[pl.ds(s * rpt + i * CHUNK,
                                                   CHUNK)])
        plsc.subcore_barrier()
        pltpu.sync_copy(dst_hbm.at[wid], idx_v)

        def body(j, carry):
            pltpu.sync_copy(ones_v, acc_sh.at[idx_v.at[j]], add=True)
            return carry

        lax.fori_loop(0, chunks, body, 0)
        plsc.subcore_barrier()
        for i in range(nzc):
            sl = pl.ds(s * rpt + i * CHUNK, CHUNK)
            pltpu.sync_copy(acc_sh.at[sl], stg_v)

            @pl.when(c == 0)
            def _():
                pltpu.sync_copy(stg_v, out0.at[sl])

            @pl.when(c == 1)
            def _():
                pltpu.sync_copy(stg_v, out1.at[sl])

    return deg_kernel


def _sc_rows(n_pad, chunks, h):
    rpt = n_pad // NS
    nzc = rpt // CHUNK
    nblocks = chunks // SB

    @functools.partial(
        pl.kernel,
        out_type=jax.ShapeDtypeStruct((NC, n_pad, h), jnp.float32),
        mesh=_mesh(),
        scratch_types=[
            pltpu.VMEM((2, 2 * SB, CHUNK), jnp.int32),
            pltpu.VMEM((CHUNK, h), jnp.float32),
            pltpu.VMEM((CHUNK, h), jnp.float32),
            pltpu.VMEM_SHARED((n_pad, h), jnp.float32),
            pltpu.SemaphoreType.DMA((2,)),
            pltpu.SemaphoreType.DMA,
            pltpu.SemaphoreType.DMA,
        ],
    )
    def rows_kernel(y_hbm, comb_hbm, zeros_hbm, out_hbm,
                    ring, d0, d1, acc_sh, isem, gsem0, gsem1):
        dbufs = (d0, d1)
        gsems = (gsem0, gsem1)
        c = lax.axis_index("c")
        s = lax.axis_index("s")
        wid = s * NC + c
        pltpu.sync_copy(zeros_hbm, d0)
        for i in range(nzc):
            pltpu.sync_copy(d0,
                            acc_sh.at[pl.ds(s * rpt + i * CHUNK, CHUNK)])
        plsc.subcore_barrier()
        pltpu.sync_copy(comb_hbm.at[wid, pl.ds(0, 2 * SB)], ring.at[0])
        pltpu.async_copy(comb_hbm.at[wid, pl.ds(2 * SB, 2 * SB)],
                         ring.at[1], isem.at[1])
        pltpu.async_copy(y_hbm.at[ring.at[0, 0]], d0, gsem0)
        pltpu.async_copy(y_hbm.at[ring.at[0, 2]], d1, gsem1)

        def outer(o, carry):
            r = o % 2
            rn = (o + 1) % 2
            for i in range(SB):
                j = 4 * o + i
                d = i % 2
                pltpu.make_async_copy(y_hbm.at[ring.at[r, 2 * i]],
                                      dbufs[d], gsems[d]).wait()
                pltpu.sync_copy(dbufs[d],
                                acc_sh.at[ring.at[r, 2 * i + 1]], add=True)

                @pl.when(j + 2 < chunks)
                def _():
                    if i < 2:
                        pltpu.async_copy(
                            y_hbm.at[ring.at[r, 2 * (i + 2)]],
                            dbufs[d], gsems[d])
                    else:
                        if i == 2:
                            pltpu.make_async_copy(
                                comb_hbm.at[wid,
                                            pl.ds(2 * SB * (o + 1), 2 * SB)],
                                ring.at[rn], isem.at[rn]).wait()
                        pltpu.async_copy(
                            y_hbm.at[ring.at[rn, 2 * (i - 2)]],
                            dbufs[d], gsems[d])

            @pl.when(o + 2 < nblocks)
            def _():
                pltpu.async_copy(
                    comb_hbm.at[wid, pl.ds(2 * SB * (o + 2), 2 * SB)],
                    ring.at[r], isem.at[r])
            return carry

        lax.fori_loop(0, nblocks, outer, 0)
        plsc.subcore_barrier()
        for i in range(nzc):
            sl = pl.ds(s * rpt + i * CHUNK, CHUNK)
            pltpu.sync_copy(acc_sh.at[sl], d0)
            pltpu.sync_copy(d0, out_hbm.at[c, sl])

    return rows_kernel


def _sc_scalar(n_pad, chunks):
    rpt = n_pad // NS
    nzc = rpt // CHUNK
    niter = chunks // K

    @functools.partial(
        pl.kernel,
        out_type=[jax.ShapeDtypeStruct((n_pad,), jnp.float32),
                  jax.ShapeDtypeStruct((n_pad,), jnp.float32)],
        mesh=_mesh(),
        scratch_types=(
            [pltpu.VMEM((chunks, CHUNK), jnp.int32),
             pltpu.VMEM((chunks, CHUNK), jnp.int32)]
            + [pltpu.VMEM((CHUNK,), jnp.float32)] * K
            + [pltpu.VMEM_SHARED((n_pad,), jnp.float32)]
            + [pltpu.SemaphoreType.DMA] * K
        ),
    )
    def scal_kernel(y_hbm, src_hbm, dst_hbm, out0, out1,
                    src_v, dst_v, *bufsem):
        bufs = bufsem[:K]
        acc_sh = bufsem[K]
        gsem = bufsem[K + 1:]
        c = lax.axis_index("c")
        s = lax.axis_index("s")
        wid = s * NC + c
        for k in range(CHUNK // 16):
            bufs[0][pl.ds(k * 16, 16)] = jnp.zeros((16,), jnp.float32)
        for i in range(nzc):
            pltpu.sync_copy(bufs[0],
                            acc_sh.at[pl.ds(s * rpt + i * CHUNK, CHUNK)])
        plsc.subcore_barrier()
        pltpu.sync_copy(src_hbm.at[wid], src_v)
        pltpu.sync_copy(dst_hbm.at[wid], dst_v)

        for b in range(K):
            pltpu.async_copy(y_hbm.at[src_v.at[b]], bufs[b], gsem[b])

        def body(o, carry):
            for b in range(K):
                j = o * K + b
                pltpu.make_async_copy(y_hbm.at[src_v.at[j]],
                                      bufs[b], gsem[b]).wait()
                pltpu.sync_copy(bufs[b], acc_sh.at[dst_v.at[j]], add=True)

                @pl.when(o < niter - 1)
                def _():
                    pltpu.async_copy(y_hbm.at[src_v.at[j + K]],
                                     bufs[b], gsem[b])
            return carry

        lax.fori_loop(0, niter, body, 0)
        plsc.subcore_barrier()
        for i in range(nzc):
            sl = pl.ds(s * rpt + i * CHUNK, CHUNK)
            pltpu.sync_copy(acc_sh.at[sl], bufs[0])

            @pl.when(c == 0)
            def _():
                pltpu.sync_copy(bufs[0], out0.at[sl])

            @pl.when(c == 1)
            def _():
                pltpu.sync_copy(bufs[0], out1.at[sl])

    return scal_kernel


def _tc_prep(n_pad, f_in, h):
    def body(deg0_ref, deg1_ref, x_ref, w_ref, y_ref, dinv_ref):
        total = deg0_ref[...] + deg1_ref[...] + 1.0
        dinv = lax.rsqrt(total)
        dinv_ref[...] = dinv
        y_ref[...] = jnp.dot(x_ref[...], w_ref[...],
                             preferred_element_type=jnp.float32) * dinv

    return pl.pallas_call(
        body,
        out_shape=[
            jax.ShapeDtypeStruct((n_pad, h), jnp.float32),
            jax.ShapeDtypeStruct((n_pad, 1), jnp.float32),
        ],
    )


def _tc_mid(n, n_pad, h, w_out):
    def body(s_ref, y_ref, dinv_ref, b_ref, g_ref, be_ref, w_ref, out_ref):
        dinv = dinv_ref[...]
        pre = dinv * (s_ref[0] + s_ref[1] + y_ref[...]) + b_ref[...]
        row = lax.broadcasted_iota(jnp.int32, (n_pad, 1), 0)
        mask = row < n
        pm = jnp.where(mask, pre, 0.0)
        mean = jnp.sum(pm, axis=0, keepdims=True) * (1.0 / n)
        meansq = jnp.sum(pm * pre, axis=0, keepdims=True) * (1.0 / n)
        var = meansq - mean * mean
        hh = (pre - mean) * lax.rsqrt(var + 1e-5) * g_ref[...] + be_ref[...]
        hh = jnp.where(mask, jnp.maximum(hh, 0.0), 0.0)
        out_ref[...] = jnp.dot(hh, w_ref[...],
                               preferred_element_type=jnp.float32) * dinv

    return pl.pallas_call(
        body,
        out_shape=jax.ShapeDtypeStruct((n_pad, w_out), jnp.float32),
    )


def _tc_final(n_pad):
    def body(s0_ref, s1_ref, y_ref, dinv_ref, b_ref, out_ref):
        out_ref[...] = (dinv_ref[...] * (s0_ref[...] + s1_ref[...]
                                         + y_ref[...]) + b_ref[...])

    return pl.pallas_call(
        body,
        out_shape=jax.ShapeDtypeStruct((n_pad, 1), jnp.float32),
    )


def kernel(x, edge_index, W1, b1, g1, be1, W2, b2, g2, be2, W3, b3):
    n, f_in = x.shape
    h = W1.shape[1]
    e = edge_index.shape[1]

    cpw = -(-e // (NW * CHUNK))
    chunks = -(-cpw // SB) * SB
    e_pad = NW * chunks * CHUNK
    n_pad = -(-(n + 1) // (NS * CHUNK)) * (NS * CHUNK)

    trash = n + jnp.arange(e_pad - e, dtype=jnp.int32) % (n_pad - n)
    src = jnp.concatenate(
        [edge_index[0].astype(jnp.int32), trash]).reshape(NW, chunks, CHUNK)
    dst = jnp.concatenate(
        [edge_index[1].astype(jnp.int32), trash]).reshape(NW, chunks, CHUNK)
    comb = jnp.stack([src, dst], axis=2).reshape(NW, 2 * chunks, CHUNK)
    x_p = jnp.zeros((n_pad, f_in), jnp.float32).at[:n].set(x)
    zeros2 = jnp.zeros((CHUNK, h), jnp.float32)

    deg0, deg1 = _sc_degree(n_pad, chunks)(dst)
    y1, dinv = _tc_prep(n_pad, f_in, h)(
        deg0.reshape(n_pad, 1), deg1.reshape(n_pad, 1), x_p, W1)

    s1 = _sc_rows(n_pad, chunks, h)(y1, comb, zeros2)
    y2 = _tc_mid(n, n_pad, h, h)(
        s1, y1, dinv, b1.reshape(1, h), g1.reshape(1, h),
        be1.reshape(1, h), W2)

    s2 = _sc_rows(n_pad, chunks, h)(y2, comb, zeros2)
    y3 = _tc_mid(n, n_pad, h, 1)(
        s2, y2, dinv, b2.reshape(1, h), g2.reshape(1, h),
        be2.reshape(1, h), W3)

    s3_0, s3_1 = _sc_scalar(n_pad, chunks)(y3.reshape(n_pad), src, dst)
    out = _tc_final(n_pad)(
        s3_0.reshape(n_pad, 1), s3_1.reshape(n_pad, 1), y3, dinv,
        b3.reshape(1, 1))
    return out[:n]


# speedup vs baseline: 2.7501x; 1.0027x over previous
import functools

import jax
import jax.numpy as jnp
from jax import lax
from jax.experimental import pallas as pl
from jax.experimental.pallas import tpu as pltpu
from jax.experimental.pallas import tpu_sc as plsc

NC = 2
NS = 16
NW = NC * NS
CHUNK = 128
K = 4
SB = 4


def _mesh():
    return plsc.VectorSubcoreMesh(core_axis_name="c", subcore_axis_name="s")


def _sc_degree(n_pad, chunks):
    rpt = n_pad // NS
    nzc = rpt // CHUNK

    @functools.partial(
        pl.kernel,
        out_type=[jax.ShapeDtypeStruct((n_pad,), jnp.float32),
                  jax.ShapeDtypeStruct((n_pad,), jnp.float32)],
        mesh=_mesh(),
        scratch_types=[
            pltpu.VMEM((chunks, CHUNK), jnp.int32),
            pltpu.VMEM((CHUNK,), jnp.float32),
            pltpu.VMEM((CHUNK,), jnp.float32),
            pltpu.VMEM_SHARED((n_pad,), jnp.float32),
        ],
    )
    def deg_kernel(dst_hbm, out0, out1, idx_v, ones_v, stg_v, acc_sh):
        c = lax.axis_index("c")
        s = lax.axis_index("s")
        wid = s * NC + c
        for k in range(CHUNK // 16):
            ones_v[pl.ds(k * 16, 16)] = jnp.ones((16,), jnp.float32)
            stg_v[pl.ds(k * 16, 16)] = jnp.zeros((16,), jnp.float32)
        for i in range(nzc):
            pltpu.sync_copy(stg_v, acc_sh.at[pl.ds(s * rpt + i * CHUNK,
                                                   CHUNK)])
        plsc.subcore_barrier()
        pltpu.sync_copy(dst_hbm.at[wid], idx_v)

        def body(j, carry):
            pltpu.sync_copy(ones_v, acc_sh.at[idx_v.at[j]], add=True)
            return carry

        lax.fori_loop(0, chunks, body, 0)
        plsc.subcore_barrier()
        for i in range(nzc):
            sl = pl.ds(s * rpt + i * CHUNK, CHUNK)
            pltpu.sync_copy(acc_sh.at[sl], stg_v)

            @pl.when(c == 0)
            def _():
                pltpu.sync_copy(stg_v, out0.at[sl])

            @pl.when(c == 1)
            def _():
                pltpu.sync_copy(stg_v, out1.at[sl])

    return deg_kernel


def _sc_rows(n_pad, chunks, h):
    rpt = n_pad // NS
    nzc = rpt // CHUNK
    nblocks = chunks // SB

    @functools.partial(
        pl.kernel,
        out_type=jax.ShapeDtypeStruct((NC, n_pad, h), jnp.float32),
        mesh=_mesh(),
        scratch_types=[
            pltpu.VMEM((2 * SB, CHUNK), jnp.int32),
            pltpu.VMEM((2 * SB, CHUNK), jnp.int32),
            pltpu.VMEM((CHUNK, h), jnp.float32),
            pltpu.VMEM((CHUNK, h), jnp.float32),
            pltpu.VMEM_SHARED((n_pad, h), jnp.float32),
            pltpu.SemaphoreType.DMA,
            pltpu.SemaphoreType.DMA,
            pltpu.SemaphoreType.DMA,
            pltpu.SemaphoreType.DMA,
        ],
    )
    def rows_kernel(y_hbm, comb_hbm, zeros_hbm, out_hbm,
                    ring0, ring1, d0, d1, acc_sh,
                    isem0, isem1, gsem0, gsem1):
        rings = (ring0, ring1)
        isems = (isem0, isem1)
        dbufs = (d0, d1)
        gsems = (gsem0, gsem1)
        c = lax.axis_index("c")
        s = lax.axis_index("s")
        wid = s * NC + c
        pltpu.sync_copy(zeros_hbm, d0)
        for i in range(nzc):
            pltpu.sync_copy(d0,
                            acc_sh.at[pl.ds(s * rpt + i * CHUNK, CHUNK)])
        plsc.subcore_barrier()
        pltpu.sync_copy(comb_hbm.at[wid, pl.ds(0, 2 * SB)], ring0)
        pltpu.async_copy(comb_hbm.at[wid, pl.ds(2 * SB, 2 * SB)],
                         ring1, isem1)
        pltpu.async_copy(y_hbm.at[ring0.at[0]], d0, gsem0)
        pltpu.async_copy(y_hbm.at[ring0.at[2]], d1, gsem1)

        def outer(m, carry):
            for r in range(2):
                o = 2 * m + r
                ring, ringn = rings[r], rings[1 - r]
                for i in range(SB):
                    j = 4 * o + i
                    d = i % 2
                    pltpu.make_async_copy(y_hbm.at[ring.at[2 * i]],
                                          dbufs[d], gsems[d]).wait()
                    pltpu.sync_copy(dbufs[d],
                                    acc_sh.at[ring.at[2 * i + 1]],
                                    add=True)

                    @pl.when(j + 2 < chunks)
                    def _():
                        if i < 2:
                            pltpu.async_copy(
                                y_hbm.at[ring.at[2 * (i + 2)]],
                                dbufs[d], gsems[d])
                        else:
                            if i == 2:
                                pltpu.make_async_copy(
                                    comb_hbm.at[
                                        wid,
                                        pl.ds(2 * SB * (o + 1), 2 * SB)],
                                    ringn, isems[1 - r]).wait()
                            pltpu.async_copy(
                                y_hbm.at[ringn.at[2 * (i - 2)]],
                                dbufs[d], gsems[d])

                @pl.when(o + 2 < nblocks)
                def _():
                    pltpu.async_copy(
                        comb_hbm.at[wid, pl.ds(2 * SB * (o + 2), 2 * SB)],
                        ring, isems[r])
            return carry

        lax.fori_loop(0, nblocks // 2, outer, 0)
        plsc.subcore_barrier()
        for i in range(nzc):
            sl = pl.ds(s * rpt + i * CHUNK, CHUNK)
            pltpu.sync_copy(acc_sh.at[sl], d0)
            pltpu.sync_copy(d0, out_hbm.at[c, sl])

    return rows_kernel


def _sc_scalar(n_pad, chunks):
    rpt = n_pad // NS
    nzc = rpt // CHUNK
    niter = chunks // K

    @functools.partial(
        pl.kernel,
        out_type=[jax.ShapeDtypeStruct((n_pad,), jnp.float32),
                  jax.ShapeDtypeStruct((n_pad,), jnp.float32)],
        mesh=_mesh(),
        scratch_types=(
            [pltpu.VMEM((chunks, CHUNK), jnp.int32),
             pltpu.VMEM((chunks, CHUNK), jnp.int32)]
            + [pltpu.VMEM((CHUNK,), jnp.float32)] * K
            + [pltpu.VMEM_SHARED((n_pad,), jnp.float32)]
            + [pltpu.SemaphoreType.DMA] * K
        ),
    )
    def scal_kernel(y_hbm, src_hbm, dst_hbm, out0, out1,
                    src_v, dst_v, *bufsem):
        bufs = bufsem[:K]
        acc_sh = bufsem[K]
        gsem = bufsem[K + 1:]
        c = lax.axis_index("c")
        s = lax.axis_index("s")
        wid = s * NC + c
        for k in range(CHUNK // 16):
            bufs[0][pl.ds(k * 16, 16)] = jnp.zeros((16,), jnp.float32)
        for i in range(nzc):
            pltpu.sync_copy(bufs[0],
                            acc_sh.at[pl.ds(s * rpt + i * CHUNK, CHUNK)])
        plsc.subcore_barrier()
        pltpu.sync_copy(src_hbm.at[wid], src_v)
        pltpu.sync_copy(dst_hbm.at[wid], dst_v)

        for b in range(K):
            pltpu.async_copy(y_hbm.at[src_v.at[b]], bufs[b], gsem[b])

        def body(o, carry):
            for b in range(K):
                j = o * K + b
                pltpu.make_async_copy(y_hbm.at[src_v.at[j]],
                                      bufs[b], gsem[b]).wait()
                pltpu.sync_copy(bufs[b], acc_sh.at[dst_v.at[j]], add=True)

                @pl.when(o < niter - 1)
                def _():
                    pltpu.async_copy(y_hbm.at[src_v.at[j + K]],
                                     bufs[b], gsem[b])
            return carry

        lax.fori_loop(0, niter, body, 0)
        plsc.subcore_barrier()
        for i in range(nzc):
            sl = pl.ds(s * rpt + i * CHUNK, CHUNK)
            pltpu.sync_copy(acc_sh.at[sl], bufs[0])

            @pl.when(c == 0)
            def _():
                pltpu.sync_copy(bufs[0], out0.at[sl])

            @pl.when(c == 1)
            def _():
                pltpu.sync_copy(bufs[0], out1.at[sl])

    return scal_kernel


def _tc_prep(n_pad, f_in, h):
    def body(deg0_ref, deg1_ref, x_ref, w_ref, y_ref, dinv_ref):
        total = deg0_ref[...] + deg1_ref[...] + 1.0
        dinv = lax.rsqrt(total)
        dinv_ref[...] = dinv
        y_ref[...] = jnp.dot(x_ref[...], w_ref[...],
                             preferred_element_type=jnp.float32) * dinv

    return pl.pallas_call(
        body,
        out_shape=[
            jax.ShapeDtypeStruct((n_pad, h), jnp.float32),
            jax.ShapeDtypeStruct((n_pad, 1), jnp.float32),
        ],
    )


def _tc_mid(n, n_pad, h, w_out):
    def body(s_ref, y_ref, dinv_ref, b_ref, g_ref, be_ref, w_ref, out_ref):
        dinv = dinv_ref[...]
        pre = dinv * (s_ref[0] + s_ref[1] + y_ref[...]) + b_ref[...]
        row = lax.broadcasted_iota(jnp.int32, (n_pad, 1), 0)
        mask = row < n
        pm = jnp.where(mask, pre, 0.0)
        mean = jnp.sum(pm, axis=0, keepdims=True) * (1.0 / n)
        meansq = jnp.sum(pm * pre, axis=0, keepdims=True) * (1.0 / n)
        var = meansq - mean * mean
        hh = (pre - mean) * lax.rsqrt(var + 1e-5) * g_ref[...] + be_ref[...]
        hh = jnp.where(mask, jnp.maximum(hh, 0.0), 0.0)
        out_ref[...] = jnp.dot(hh, w_ref[...],
                               preferred_element_type=jnp.float32) * dinv

    return pl.pallas_call(
        body,
        out_shape=jax.ShapeDtypeStruct((n_pad, w_out), jnp.float32),
    )


def _tc_final(n_pad):
    def body(s0_ref, s1_ref, y_ref, dinv_ref, b_ref, out_ref):
        out_ref[...] = (dinv_ref[...] * (s0_ref[...] + s1_ref[...]
                                         + y_ref[...]) + b_ref[...])

    return pl.pallas_call(
        body,
        out_shape=jax.ShapeDtypeStruct((n_pad, 1), jnp.float32),
    )


def kernel(x, edge_index, W1, b1, g1, be1, W2, b2, g2, be2, W3, b3):
    n, f_in = x.shape
    h = W1.shape[1]
    e = edge_index.shape[1]

    cpw = -(-e // (NW * CHUNK))
    chunks = -(-cpw // (2 * SB)) * (2 * SB)
    e_pad = NW * chunks * CHUNK
    n_pad = -(-(n + 1) // (NS * CHUNK)) * (NS * CHUNK)

    trash = n + jnp.arange(e_pad - e, dtype=jnp.int32) % (n_pad - n)
    src = jnp.concatenate(
        [edge_index[0].astype(jnp.int32), trash]).reshape(NW, chunks, CHUNK)
    dst = jnp.concatenate(
        [edge_index[1].astype(jnp.int32), trash]).reshape(NW, chunks, CHUNK)
    comb = jnp.stack([src, dst], axis=2).reshape(NW, 2 * chunks, CHUNK)
    x_p = jnp.zeros((n_pad, f_in), jnp.float32).at[:n].set(x)
    zeros2 = jnp.zeros((CHUNK, h), jnp.float32)

    deg0, deg1 = _sc_degree(n_pad, chunks)(dst)
    y1, dinv = _tc_prep(n_pad, f_in, h)(
        deg0.reshape(n_pad, 1), deg1.reshape(n_pad, 1), x_p, W1)

    s1 = _sc_rows(n_pad, chunks, h)(y1, comb, zeros2)
    y2 = _tc_mid(n, n_pad, h, h)(
        s1, y1, dinv, b1.reshape(1, h), g1.reshape(1, h),
        be1.reshape(1, h), W2)

    s2 = _sc_rows(n_pad, chunks, h)(y2, comb, zeros2)
    y3 = _tc_mid(n, n_pad, h, 1)(
        s2, y2, dinv, b2.reshape(1, h), g2.reshape(1, h),
        be2.reshape(1, h), W3)

    s3_0, s3_1 = _sc_scalar(n_pad, chunks)(y3.reshape(n_pad), src, dst)
    out = _tc_final(n_pad)(
        s3_0.reshape(n_pad, 1), s3_1.reshape(n_pad, 1), y3, dinv,
        b3.reshape(1, 1))
    return out[:n]
